# Initial kernel scaffold; baseline (speedup 1.0000x reference)
import jax
import jax.numpy as jnp
from jax.experimental import pallas as pl


def kernel(x_static, x_dynamic, x_prog, x_time, edge_index, W1, asrc1, adst1, bias1, W2, asrc2, adst2, bias2, W3, asrc3, adst3, bias3, W4, asrc4, adst4, bias4, W5, asrc5, adst5, bias5, fc1_W, fc1_b, fc2_W, fc2_b):
    raise NotImplementedError("write your pallas kernel here")


# speedup vs baseline: 10.6270x; 10.6270x over previous
import functools

import jax
import jax.numpy as jnp
from jax import lax
from jax.experimental import pallas as pl
from jax.experimental.pallas import tpu as pltpu
from jax.experimental.pallas import tpu_sc as plsc

N = 10000
E = 320000
ET = E + N
HID = 172
HP = 176
NW = 32
R = 320
NP = NW * R
OFS_T = R + 16
OFFP = NP + 16
SLACK = 24576
ESL = ET + SLACK
LANES = 16
KV = HP // LANES


def _mm_first_body(x_ref, w_ref, a_ref, h_ref, aa_ref):
    h = jnp.dot(x_ref[...], w_ref[...], preferred_element_type=jnp.float32)
    h_ref[...] = h
    aa_ref[...] = jnp.dot(h, a_ref[...], preferred_element_type=jnp.float32)


def _mm_mid_body(o_ref, b_ref, w_ref, a_ref, h_ref, aa_ref):
    x = jnp.maximum(o_ref[...] + b_ref[0:1, :], 0.0)
    h = jnp.dot(x, w_ref[...], preferred_element_type=jnp.float32)
    h_ref[...] = h
    aa_ref[...] = jnp.dot(h, a_ref[...], preferred_element_type=jnp.float32)


def _fc_body(o4_ref, b4_ref, w1_ref, c1_ref, o5_ref, b5_ref, w2_ref, c2_ref,
             y1_ref, y2_ref):
    x4 = jnp.maximum(o4_ref[...] + b4_ref[0:1, :], 0.0)
    y1_ref[...] = jnp.dot(x4, w1_ref[...],
                          preferred_element_type=jnp.float32) + c1_ref[0:1, :]
    x5 = jnp.maximum(o5_ref[...] + b5_ref[0:1, :], 0.0)
    y2_ref[...] = jnp.dot(x5, w2_ref[...],
                          preferred_element_type=jnp.float32) + c2_ref[0:1, :]


_BM = 1024


def _tc_first(x, w, a2):
    din = x.shape[1]
    return pl.pallas_call(
        _mm_first_body,
        grid=(NP // _BM,),
        in_specs=[
            pl.BlockSpec((_BM, din), lambda i: (i, 0)),
            pl.BlockSpec((din, HP), lambda i: (0, 0)),
            pl.BlockSpec((HP, 8), lambda i: (0, 0)),
        ],
        out_specs=[
            pl.BlockSpec((_BM, HP), lambda i: (i, 0)),
            pl.BlockSpec((_BM, 8), lambda i: (i, 0)),
        ],
        out_shape=[
            jax.ShapeDtypeStruct((NP, HP), jnp.float32),
            jax.ShapeDtypeStruct((NP, 8), jnp.float32),
        ],
    )(x, w, a2)


def _tc_mid(o_prev, b_prev, w, a2):
    return pl.pallas_call(
        _mm_mid_body,
        grid=(NP // _BM,),
        in_specs=[
            pl.BlockSpec((_BM, HP), lambda i: (i, 0)),
            pl.BlockSpec((8, HP), lambda i: (0, 0)),
            pl.BlockSpec((HP, HP), lambda i: (0, 0)),
            pl.BlockSpec((HP, 8), lambda i: (0, 0)),
        ],
        out_specs=[
            pl.BlockSpec((_BM, HP), lambda i: (i, 0)),
            pl.BlockSpec((_BM, 8), lambda i: (i, 0)),
        ],
        out_shape=[
            jax.ShapeDtypeStruct((NP, HP), jnp.float32),
            jax.ShapeDtypeStruct((NP, 8), jnp.float32),
        ],
    )(o_prev, b_prev, w, a2)


def _tc_head(o4, b4, w1, c1, o5, b5, w2, c2):
    return pl.pallas_call(
        _fc_body,
        grid=(NP // _BM,),
        in_specs=[
            pl.BlockSpec((_BM, HP), lambda i: (i, 0)),
            pl.BlockSpec((8, HP), lambda i: (0, 0)),
            pl.BlockSpec((HP, 128), lambda i: (0, 0)),
            pl.BlockSpec((8, 128), lambda i: (0, 0)),
            pl.BlockSpec((_BM, HP), lambda i: (i, 0)),
            pl.BlockSpec((8, HP), lambda i: (0, 0)),
            pl.BlockSpec((HP, 128), lambda i: (0, 0)),
            pl.BlockSpec((8, 128), lambda i: (0, 0)),
        ],
        out_specs=[
            pl.BlockSpec((_BM, 128), lambda i: (i, 0)),
            pl.BlockSpec((_BM, 128), lambda i: (i, 0)),
        ],
        out_shape=[
            jax.ShapeDtypeStruct((NP, 128), jnp.float32),
            jax.ShapeDtypeStruct((NP, 128), jnp.float32),
        ],
    )(o4, b4, w1, c1, o5, b5, w2, c2)


def _hmax(v):
    r = v[0]
    for j in range(1, LANES):
        r = jnp.maximum(r, v[j])
    return r


def _hsum(v):
    r = v[0]
    for j in range(1, LANES):
        r = r + v[j]
    return r


def _gat_edge_body(h_hbm, as_hbm, ad_hbm, src_hbm, offs_hbm, out_hbm,
              as_buf, ad_buf, src_buf, offs_buf, out_buf, h_stage, sem):
    wid = lax.axis_index("s") * 2 + lax.axis_index("c")
    r0 = wid * R
    pltpu.sync_copy(as_hbm, as_buf)
    pltpu.sync_copy(ad_hbm.at[pl.ds(r0, OFS_T)], ad_buf)
    pltpu.sync_copy(offs_hbm.at[pl.ds(r0, OFS_T)], offs_buf)
    e0 = offs_buf[pl.ds(0, LANES)][0]
    a_lo = pl.multiple_of(e0 & ~7, 8)
    pltpu.sync_copy(src_hbm.at[pl.ds(a_lo, SLACK)], src_buf)

    def _max_body(i, m):
        return jnp.maximum(m, as_buf[pl.ds(i * LANES, LANES)])

    m0 = lax.fori_loop(0, NP // LANES, _max_body,
                       jnp.full((LANES,), -3e38, jnp.float32))
    amax = _hmax(m0)

    iota = lax.iota(jnp.int32, LANES)
    zero = jnp.zeros((LANES,), jnp.float32)

    def _row_body(rr, _):
        ov = offs_buf[pl.ds(rr, LANES)]
        e_lo = ov[0]
        e_hi = ov[1]
        ad_r = ad_buf[pl.ds(rr, LANES)][0]
        t = amax + ad_r
        shift = jnp.maximum(t, 0.2 * t)
        nch = (e_hi - e_lo + (LANES - 1)) >> 4

        def _chunk_body(ci, carry):
            den = carry[0]
            accs = list(carry[1:])
            gbase = e_lo + ci * LANES
            idxv = jnp.minimum(gbase - a_lo + iota, SLACK - 1)
            sidx = plsc.load_gather(src_buf, [idxv])
            sidx = jnp.clip(sidx, 0, NP - 1)
            av = plsc.load_gather(as_buf, [sidx])
            t2 = av + ad_r
            lg = jnp.maximum(t2, 0.2 * t2)
            ex = jnp.where(gbase + iota < e_hi,
                           jnp.exp(lg - shift), 0.0)
            pltpu.async_copy(h_hbm.at[sidx], h_stage, sem).wait()
            for j in range(LANES):
                cv = jnp.full((LANES,), ex[j], jnp.float32)
                for k in range(KV):
                    accs[k] = accs[k] + cv * h_stage[j, pl.ds(k * LANES, LANES)]
            return (den + ex, *accs)

        init = (zero,) + tuple(zero for _ in range(KV))
        res = lax.fori_loop(0, nch, _chunk_body, init)
        denv = jnp.full((LANES,), _hsum(res[0]), jnp.float32) + 1e-16
        rinv = jnp.ones((LANES,), jnp.float32) / denv
        for k in range(KV):
            out_buf[pl.ds(rr * HP + k * LANES, LANES)] = res[1 + k] * rinv
        return 0

    lax.fori_loop(0, R, _row_body, 0)
    pltpu.sync_copy(out_buf, out_hbm.at[pl.ds(r0 * HP, R * HP)])


_gat_edge_built = None


def _gat_edge(*args):
    global _gat_edge_built
    if _gat_edge_built is None:
        mesh = plsc.VectorSubcoreMesh(core_axis_name="c", subcore_axis_name="s",
                                      num_cores=2, num_subcores=16)
        _gat_edge_built = functools.partial(
            pl.kernel,
            out_type=jax.ShapeDtypeStruct((NP * HP,), jnp.float32),
            mesh=mesh,
            compiler_params=pltpu.CompilerParams(needs_layout_passes=False,
                                                 use_tc_tiling_on_sc=False),
            scratch_types=[
                pltpu.VMEM((NP,), jnp.float32),
                pltpu.VMEM((OFS_T,), jnp.float32),
                pltpu.VMEM((SLACK,), jnp.int32),
                pltpu.VMEM((OFS_T,), jnp.int32),
                pltpu.VMEM((R * HP,), jnp.float32),
                pltpu.VMEM((LANES, HP), jnp.float32),
                pltpu.SemaphoreType.DMA,
            ],
        )(_gat_edge_body)
    return _gat_edge_built(*args)


def _pad_w(w):
    return jnp.zeros((HP, HP), jnp.float32).at[:w.shape[0], :w.shape[1]].set(w)


def _pad_a2(a_s, a_d):
    a2 = jnp.zeros((HP, 8), jnp.float32)
    return a2.at[:HID, 0].set(a_s).at[:HID, 1].set(a_d)


def _pad_b(b):
    return jnp.tile(jnp.pad(b, (0, HP - HID))[None, :], (8, 1))


def kernel(x_static, x_dynamic, x_prog, x_time, edge_index,
           W1, asrc1, adst1, bias1, W2, asrc2, adst2, bias2,
           W3, asrc3, adst3, bias3, W4, asrc4, adst4, bias4,
           W5, asrc5, adst5, bias5, fc1_W, fc1_b, fc2_W, fc2_b):
    f32 = jnp.float32
    x0 = jnp.concatenate(
        [x_static, x_dynamic, x_prog, x_time.astype(f32)], axis=-1)
    din0 = 48
    x0 = jnp.pad(x0, ((0, NP - N), (0, din0 - x0.shape[1])))
    src, dst = edge_index[0], edge_index[1]
    loop = jnp.arange(N, dtype=src.dtype)
    d_all = jnp.concatenate([dst, loop])
    s_all = jnp.concatenate([src, loop])
    d_s, s_s = lax.sort((d_all, s_all), num_keys=1)
    offs = jnp.searchsorted(
        d_s, jnp.arange(OFFP, dtype=jnp.int32), side='left').astype(jnp.int32)
    s_pad = jnp.pad(s_s.astype(jnp.int32), (0, SLACK))

    w1p = jnp.zeros((din0, HP), f32).at[:x_static.shape[1] + x_dynamic.shape[1]
                                        + x_prog.shape[1] + x_time.shape[1],
                                        :HID].set(W1)
    layers = [
        (w1p, _pad_a2(asrc1, adst1), None),
        (_pad_w(W2), _pad_a2(asrc2, adst2), _pad_b(bias1)),
        (_pad_w(W3), _pad_a2(asrc3, adst3), _pad_b(bias2)),
        (_pad_w(W4), _pad_a2(asrc4, adst4), _pad_b(bias3)),
        (_pad_w(W5), _pad_a2(asrc5, adst5), _pad_b(bias3)),
    ]

    def run_layer(x_or_o, w, a2, b_prev, first):
        if first:
            h, aa = _tc_first(x_or_o, w, a2)
        else:
            h, aa = _tc_mid(x_or_o, b_prev, w, a2)
        adv = jnp.pad(aa[:, 1], (0, OFS_T))
        out_flat = _gat_edge(h, aa[:, 0], adv, s_pad, offs)
        return out_flat.reshape(NP, HP)

    o1 = run_layer(x0, *layers[0][:2], None, True)
    o2 = run_layer(o1, *layers[1][:2], layers[1][2], False)
    o3 = run_layer(o2, *layers[2][:2], layers[2][2], False)
    o4 = run_layer(o3, *layers[3][:2], layers[3][2], False)
    o5 = run_layer(o3, *layers[4][:2], layers[4][2], False)

    w1h = jnp.zeros((HP, 128), f32).at[:HID, :7].set(fc1_W)
    c1h = jnp.tile(jnp.pad(fc1_b, (0, 128 - 7))[None, :], (8, 1))
    w2h = jnp.zeros((HP, 128), f32).at[:HID, :3].set(fc2_W)
    c2h = jnp.tile(jnp.pad(fc2_b, (0, 128 - 3))[None, :], (8, 1))
    y1, y2 = _tc_head(o4, _pad_b(bias4), w1h, c1h,
                      o5, _pad_b(bias5), w2h, c2h)
    zero = jnp.float32(0.0)
    return (y1[:N, :7], y2[:N, :3], zero, zero)


# speedup vs baseline: 21.0736x; 1.9830x over previous
import functools

import jax
import jax.numpy as jnp
from jax import lax
from jax.experimental import pallas as pl
from jax.experimental.pallas import tpu as pltpu
from jax.experimental.pallas import tpu_sc as plsc

N = 10000
E = 320000
ET = E + N
HID = 172
HP = 176
NW = 32
R = 320
NP = NW * R
OFS_T = R + 16
OFFP = NP + 16
SLACK = 24576
ESL = ET + SLACK
LANES = 16
KV = HP // LANES


def _mm_first_body(x_ref, w_ref, a_ref, h_ref, aa_ref):
    h = jnp.dot(x_ref[...], w_ref[...], preferred_element_type=jnp.float32)
    h_ref[...] = h
    aa_ref[...] = jnp.dot(h, a_ref[...], preferred_element_type=jnp.float32)


def _mm_mid_body(o_ref, b_ref, w_ref, a_ref, h_ref, aa_ref):
    x = jnp.maximum(o_ref[...] + b_ref[0:1, :], 0.0)
    h = jnp.dot(x, w_ref[...], preferred_element_type=jnp.float32)
    h_ref[...] = h
    aa_ref[...] = jnp.dot(h, a_ref[...], preferred_element_type=jnp.float32)


def _fc_body(o4_ref, b4_ref, w1_ref, c1_ref, o5_ref, b5_ref, w2_ref, c2_ref,
             y1_ref, y2_ref):
    x4 = jnp.maximum(o4_ref[...] + b4_ref[0:1, :], 0.0)
    y1_ref[...] = jnp.dot(x4, w1_ref[...],
                          preferred_element_type=jnp.float32) + c1_ref[0:1, :]
    x5 = jnp.maximum(o5_ref[...] + b5_ref[0:1, :], 0.0)
    y2_ref[...] = jnp.dot(x5, w2_ref[...],
                          preferred_element_type=jnp.float32) + c2_ref[0:1, :]


_BM = 1024


def _tc_first(x, w, a2):
    din = x.shape[1]
    return pl.pallas_call(
        _mm_first_body,
        grid=(NP // _BM,),
        in_specs=[
            pl.BlockSpec((_BM, din), lambda i: (i, 0)),
            pl.BlockSpec((din, HP), lambda i: (0, 0)),
            pl.BlockSpec((HP, 8), lambda i: (0, 0)),
        ],
        out_specs=[
            pl.BlockSpec((_BM, HP), lambda i: (i, 0)),
            pl.BlockSpec((_BM, 8), lambda i: (i, 0)),
        ],
        out_shape=[
            jax.ShapeDtypeStruct((NP, HP), jnp.float32),
            jax.ShapeDtypeStruct((NP, 8), jnp.float32),
        ],
    )(x, w, a2)


def _tc_mid(o_prev, b_prev, w, a2):
    return pl.pallas_call(
        _mm_mid_body,
        grid=(NP // _BM,),
        in_specs=[
            pl.BlockSpec((_BM, HP), lambda i: (i, 0)),
            pl.BlockSpec((8, HP), lambda i: (0, 0)),
            pl.BlockSpec((HP, HP), lambda i: (0, 0)),
            pl.BlockSpec((HP, 8), lambda i: (0, 0)),
        ],
        out_specs=[
            pl.BlockSpec((_BM, HP), lambda i: (i, 0)),
            pl.BlockSpec((_BM, 8), lambda i: (i, 0)),
        ],
        out_shape=[
            jax.ShapeDtypeStruct((NP, HP), jnp.float32),
            jax.ShapeDtypeStruct((NP, 8), jnp.float32),
        ],
    )(o_prev, b_prev, w, a2)


def _tc_head(o4, b4, w1, c1, o5, b5, w2, c2):
    return pl.pallas_call(
        _fc_body,
        grid=(NP // _BM,),
        in_specs=[
            pl.BlockSpec((_BM, HP), lambda i: (i, 0)),
            pl.BlockSpec((8, HP), lambda i: (0, 0)),
            pl.BlockSpec((HP, 128), lambda i: (0, 0)),
            pl.BlockSpec((8, 128), lambda i: (0, 0)),
            pl.BlockSpec((_BM, HP), lambda i: (i, 0)),
            pl.BlockSpec((8, HP), lambda i: (0, 0)),
            pl.BlockSpec((HP, 128), lambda i: (0, 0)),
            pl.BlockSpec((8, 128), lambda i: (0, 0)),
        ],
        out_specs=[
            pl.BlockSpec((_BM, 128), lambda i: (i, 0)),
            pl.BlockSpec((_BM, 128), lambda i: (i, 0)),
        ],
        out_shape=[
            jax.ShapeDtypeStruct((NP, 128), jnp.float32),
            jax.ShapeDtypeStruct((NP, 128), jnp.float32),
        ],
    )(o4, b4, w1, c1, o5, b5, w2, c2)


def _hmax(v):
    r = v[0]
    for j in range(1, LANES):
        r = jnp.maximum(r, v[j])
    return r


def _hsum(v):
    r = v[0]
    for j in range(1, LANES):
        r = r + v[j]
    return r


NBUF = 4


def _gat_edge_body(h_hbm, as_hbm, ad_hbm, src_hbm, offs_hbm, out_hbm,
                   as_buf, ad_buf, src_buf, offs_buf, out_buf, den_buf,
                   h_stage, sem):
    wid = lax.axis_index("s") * 2 + lax.axis_index("c")
    r0 = wid * R
    pltpu.sync_copy(as_hbm, as_buf)
    pltpu.sync_copy(ad_hbm.at[pl.ds(r0, OFS_T)], ad_buf)
    pltpu.sync_copy(offs_hbm.at[pl.ds(r0, OFS_T)], offs_buf)
    e0 = offs_buf[pl.ds(0, LANES)][0]
    a_lo = pl.multiple_of(e0 & ~7, 8)
    pltpu.sync_copy(src_hbm.at[pl.ds(a_lo, SLACK)], src_buf)

    def _max_body(i, m):
        return jnp.maximum(m, as_buf[pl.ds(i * LANES, LANES)])

    m0 = lax.fori_loop(0, NP // LANES, _max_body,
                       jnp.full((LANES,), -3e38, jnp.float32))
    amax = _hmax(m0)

    iota = lax.iota(jnp.int32, LANES)
    zero = jnp.zeros((LANES,), jnp.float32)

    def _zero_body(i, _):
        out_buf[pl.ds(i * LANES, LANES)] = zero
        return 0

    lax.fori_loop(0, R * HP // LANES, _zero_body, 0)

    def _zero_den(i, _):
        den_buf[pl.ds(i * LANES, LANES)] = zero
        return 0

    lax.fori_loop(0, R, _zero_den, 0)

    def _advance(row, start):
        rs = jnp.minimum(row, R - 1)
        ov = offs_buf[pl.ds(rs, LANES)]
        nstart = start + LANES
        same = nstart < ov[1]
        nrow = jnp.where(same, rs, jnp.where(ov[2] > ov[1], rs + 1, R))
        nst = jnp.where(same, nstart, ov[1])
        nrow = jnp.where(row >= R, R, nrow)
        return nrow, nst

    def _chunk_sidx(start):
        idxv = jnp.minimum(jnp.maximum(start - a_lo, 0) + iota, SLACK - 1)
        sidx = plsc.load_gather(src_buf, [idxv])
        return jnp.clip(sidx, 0, NP - 1)

    def _fire(row, start, pb):
        sidx = _chunk_sidx(start)

        @pl.when(row < R)
        def _():
            pltpu.async_copy(h_hbm.at[sidx], h_stage.at[pb], sem.at[pb])

    c_row = jnp.int32(0)
    c_start = e0
    f_row = c_row
    f_start = c_start
    for i in range(NBUF - 1):
        _fire(f_row, f_start, jnp.int32(i))
        f_row, f_start = _advance(f_row, f_start)

    def _cond(carry):
        return carry[1] < R

    def _body(carry):
        cnt, c_row, c_start, f_row, f_start = carry
        _fire(f_row, f_start, (cnt + (NBUF - 1)) & (NBUF - 1))
        nf_row, nf_start = _advance(f_row, f_start)
        pc = cnt & (NBUF - 1)
        ov = offs_buf[pl.ds(c_row, LANES)]
        e_hi = ov[1]
        ad_r = ad_buf[pl.ds(c_row, LANES)][0]
        t = amax + ad_r
        shift = jnp.maximum(t, 0.2 * t)
        sidx = _chunk_sidx(c_start)
        av = plsc.load_gather(as_buf, [sidx])
        t2 = av + ad_r
        lg = jnp.maximum(t2, 0.2 * t2)
        ex = jnp.where(c_start + iota < e_hi, jnp.exp(lg - shift), 0.0)
        pltpu.make_async_copy(h_hbm.at[pl.ds(0, LANES)], h_stage.at[pc],
                              sem.at[pc]).wait()
        ob = c_row * HP
        accs = [out_buf[pl.ds(ob + k * LANES, LANES)] for k in range(KV)]
        for j in range(LANES):
            cv = jnp.full((LANES,), ex[j], jnp.float32)
            for k in range(KV):
                accs[k] = accs[k] + cv * h_stage[pc, j, pl.ds(k * LANES, LANES)]
        for k in range(KV):
            out_buf[pl.ds(ob + k * LANES, LANES)] = accs[k]
        db = c_row * LANES
        den_buf[pl.ds(db, LANES)] = den_buf[pl.ds(db, LANES)] + ex
        nc_row, nc_start = _advance(c_row, c_start)
        return (cnt + 1, nc_row, nc_start, nf_row, nf_start)

    lax.while_loop(_cond, _body,
                   (jnp.int32(0), c_row, c_start, f_row, f_start))

    def _norm_body(rr, _):
        d = _hsum(den_buf[pl.ds(rr * LANES, LANES)])
        denv = jnp.full((LANES,), d, jnp.float32) + 1e-16
        rinv = jnp.ones((LANES,), jnp.float32) / denv
        ob = rr * HP
        for k in range(KV):
            out_buf[pl.ds(ob + k * LANES, LANES)] = (
                out_buf[pl.ds(ob + k * LANES, LANES)] * rinv)
        return 0

    lax.fori_loop(0, R, _norm_body, 0)
    pltpu.sync_copy(out_buf, out_hbm.at[pl.ds(r0 * HP, R * HP)])


_gat_edge_built = None


def _gat_edge(*args):
    global _gat_edge_built
    if _gat_edge_built is None:
        mesh = plsc.VectorSubcoreMesh(core_axis_name="c", subcore_axis_name="s",
                                      num_cores=2, num_subcores=16)
        _gat_edge_built = functools.partial(
            pl.kernel,
            out_type=jax.ShapeDtypeStruct((NP * HP,), jnp.float32),
            mesh=mesh,
            compiler_params=pltpu.CompilerParams(needs_layout_passes=False,
                                                 use_tc_tiling_on_sc=False),
            scratch_types=[
                pltpu.VMEM((NP,), jnp.float32),
                pltpu.VMEM((OFS_T,), jnp.float32),
                pltpu.VMEM((SLACK,), jnp.int32),
                pltpu.VMEM((OFS_T,), jnp.int32),
                pltpu.VMEM((R * HP,), jnp.float32),
                pltpu.VMEM((R * LANES,), jnp.float32),
                pltpu.VMEM((NBUF, LANES, HP), jnp.float32),
                pltpu.SemaphoreType.DMA((NBUF,)),
            ],
        )(_gat_edge_body)
    return _gat_edge_built(*args)


def _pad_w(w):
    return jnp.zeros((HP, HP), jnp.float32).at[:w.shape[0], :w.shape[1]].set(w)


def _pad_a2(a_s, a_d):
    a2 = jnp.zeros((HP, 8), jnp.float32)
    return a2.at[:HID, 0].set(a_s).at[:HID, 1].set(a_d)


def _pad_b(b):
    return jnp.tile(jnp.pad(b, (0, HP - HID))[None, :], (8, 1))


def kernel(x_static, x_dynamic, x_prog, x_time, edge_index,
           W1, asrc1, adst1, bias1, W2, asrc2, adst2, bias2,
           W3, asrc3, adst3, bias3, W4, asrc4, adst4, bias4,
           W5, asrc5, adst5, bias5, fc1_W, fc1_b, fc2_W, fc2_b):
    f32 = jnp.float32
    x0 = jnp.concatenate(
        [x_static, x_dynamic, x_prog, x_time.astype(f32)], axis=-1)
    din0 = 48
    x0 = jnp.pad(x0, ((0, NP - N), (0, din0 - x0.shape[1])))
    src, dst = edge_index[0], edge_index[1]
    loop = jnp.arange(N, dtype=src.dtype)
    d_all = jnp.concatenate([dst, loop])
    s_all = jnp.concatenate([src, loop])
    d_s, s_s = lax.sort((d_all, s_all), num_keys=1)
    offs = jnp.searchsorted(
        d_s, jnp.arange(OFFP, dtype=jnp.int32), side='left').astype(jnp.int32)
    s_pad = jnp.pad(s_s.astype(jnp.int32), (0, SLACK))

    w1p = jnp.zeros((din0, HP), f32).at[:x_static.shape[1] + x_dynamic.shape[1]
                                        + x_prog.shape[1] + x_time.shape[1],
                                        :HID].set(W1)
    layers = [
        (w1p, _pad_a2(asrc1, adst1), None),
        (_pad_w(W2), _pad_a2(asrc2, adst2), _pad_b(bias1)),
        (_pad_w(W3), _pad_a2(asrc3, adst3), _pad_b(bias2)),
        (_pad_w(W4), _pad_a2(asrc4, adst4), _pad_b(bias3)),
        (_pad_w(W5), _pad_a2(asrc5, adst5), _pad_b(bias3)),
    ]

    def run_layer(x_or_o, w, a2, b_prev, first):
        if first:
            h, aa = _tc_first(x_or_o, w, a2)
        else:
            h, aa = _tc_mid(x_or_o, b_prev, w, a2)
        adv = jnp.pad(aa[:, 1], (0, OFS_T))
        out_flat = _gat_edge(h, aa[:, 0], adv, s_pad, offs)
        return out_flat.reshape(NP, HP)

    o1 = run_layer(x0, *layers[0][:2], None, True)
    o2 = run_layer(o1, *layers[1][:2], layers[1][2], False)
    o3 = run_layer(o2, *layers[2][:2], layers[2][2], False)
    o4 = run_layer(o3, *layers[3][:2], layers[3][2], False)
    o5 = run_layer(o3, *layers[4][:2], layers[4][2], False)

    w1h = jnp.zeros((HP, 128), f32).at[:HID, :7].set(fc1_W)
    c1h = jnp.tile(jnp.pad(fc1_b, (0, 128 - 7))[None, :], (8, 1))
    w2h = jnp.zeros((HP, 128), f32).at[:HID, :3].set(fc2_W)
    c2h = jnp.tile(jnp.pad(fc2_b, (0, 128 - 3))[None, :], (8, 1))
    y1, y2 = _tc_head(o4, _pad_b(bias4), w1h, c1h,
                      o5, _pad_b(bias5), w2h, c2h)
    zero = jnp.float32(0.0)
    return (y1[:N, :7], y2[:N, :3], zero, zero)


# speedup vs baseline: 27.1036x; 1.2861x over previous
import functools

import jax
import jax.numpy as jnp
from jax import lax
from jax.experimental import pallas as pl
from jax.experimental.pallas import tpu as pltpu
from jax.experimental.pallas import tpu_sc as plsc

N = 10000
E = 320000
ET = E + N
HID = 172
HP = 176
NW = 32
R = 320
NP = NW * R
OFS_T = R + 16
OFFP = NP + 16
SLACK = 24576
ESL = ET + SLACK
LANES = 16
KV = HP // LANES


def _mm_first_body(x_ref, w_ref, a_ref, h_ref, aa_ref):
    h = jnp.dot(x_ref[...], w_ref[...], preferred_element_type=jnp.float32)
    h_ref[...] = h
    aa_ref[...] = jnp.dot(h, a_ref[...], preferred_element_type=jnp.float32)


def _mm_mid_body(o_ref, b_ref, w_ref, a_ref, h_ref, aa_ref):
    x = jnp.maximum(o_ref[...] + b_ref[0:1, :], 0.0)
    h = jnp.dot(x, w_ref[...], preferred_element_type=jnp.float32)
    h_ref[...] = h
    aa_ref[...] = jnp.dot(h, a_ref[...], preferred_element_type=jnp.float32)


def _fc_body(o4_ref, b4_ref, w1_ref, c1_ref, o5_ref, b5_ref, w2_ref, c2_ref,
             y1_ref, y2_ref):
    x4 = jnp.maximum(o4_ref[...] + b4_ref[0:1, :], 0.0)
    y1_ref[...] = jnp.dot(x4, w1_ref[...],
                          preferred_element_type=jnp.float32) + c1_ref[0:1, :]
    x5 = jnp.maximum(o5_ref[...] + b5_ref[0:1, :], 0.0)
    y2_ref[...] = jnp.dot(x5, w2_ref[...],
                          preferred_element_type=jnp.float32) + c2_ref[0:1, :]


_BM = 1024


def _tc_first(x, w, a2):
    din = x.shape[1]
    return pl.pallas_call(
        _mm_first_body,
        grid=(NP // _BM,),
        in_specs=[
            pl.BlockSpec((_BM, din), lambda i: (i, 0)),
            pl.BlockSpec((din, HP), lambda i: (0, 0)),
            pl.BlockSpec((HP, 8), lambda i: (0, 0)),
        ],
        out_specs=[
            pl.BlockSpec((_BM, HP), lambda i: (i, 0)),
            pl.BlockSpec((_BM, 8), lambda i: (i, 0)),
        ],
        out_shape=[
            jax.ShapeDtypeStruct((NP, HP), jnp.float32),
            jax.ShapeDtypeStruct((NP, 8), jnp.float32),
        ],
    )(x, w, a2)


def _tc_mid(o_prev, b_prev, w, a2):
    return pl.pallas_call(
        _mm_mid_body,
        grid=(NP // _BM,),
        in_specs=[
            pl.BlockSpec((_BM, HP), lambda i: (i, 0)),
            pl.BlockSpec((8, HP), lambda i: (0, 0)),
            pl.BlockSpec((HP, HP), lambda i: (0, 0)),
            pl.BlockSpec((HP, 8), lambda i: (0, 0)),
        ],
        out_specs=[
            pl.BlockSpec((_BM, HP), lambda i: (i, 0)),
            pl.BlockSpec((_BM, 8), lambda i: (i, 0)),
        ],
        out_shape=[
            jax.ShapeDtypeStruct((NP, HP), jnp.float32),
            jax.ShapeDtypeStruct((NP, 8), jnp.float32),
        ],
    )(o_prev, b_prev, w, a2)


def _tc_head(o4, b4, w1, c1, o5, b5, w2, c2):
    return pl.pallas_call(
        _fc_body,
        grid=(NP // _BM,),
        in_specs=[
            pl.BlockSpec((_BM, HP), lambda i: (i, 0)),
            pl.BlockSpec((8, HP), lambda i: (0, 0)),
            pl.BlockSpec((HP, 128), lambda i: (0, 0)),
            pl.BlockSpec((8, 128), lambda i: (0, 0)),
            pl.BlockSpec((_BM, HP), lambda i: (i, 0)),
            pl.BlockSpec((8, HP), lambda i: (0, 0)),
            pl.BlockSpec((HP, 128), lambda i: (0, 0)),
            pl.BlockSpec((8, 128), lambda i: (0, 0)),
        ],
        out_specs=[
            pl.BlockSpec((_BM, 128), lambda i: (i, 0)),
            pl.BlockSpec((_BM, 128), lambda i: (i, 0)),
        ],
        out_shape=[
            jax.ShapeDtypeStruct((NP, 128), jnp.float32),
            jax.ShapeDtypeStruct((NP, 128), jnp.float32),
        ],
    )(o4, b4, w1, c1, o5, b5, w2, c2)


def _hmax(v):
    r = v[0]
    for j in range(1, LANES):
        r = jnp.maximum(r, v[j])
    return r


def _hsum(v):
    r = v[0]
    for j in range(1, LANES):
        r = r + v[j]
    return r


NBUF = 4


def _gat_edge_body(h_hbm, as_hbm, ad_hbm, src_hbm, offs_hbm, out_hbm,
                   as_buf, ad_buf, src_buf, offs_buf, out_buf, den_buf,
                   h_stage, sem):
    wid = lax.axis_index("s") * 2 + lax.axis_index("c")
    r0 = wid * R
    pltpu.sync_copy(as_hbm, as_buf)
    pltpu.sync_copy(ad_hbm.at[pl.ds(r0, OFS_T)], ad_buf)
    pltpu.sync_copy(offs_hbm.at[pl.ds(r0, OFS_T)], offs_buf)
    e0 = offs_buf[pl.ds(0, LANES)][0]
    a_lo = pl.multiple_of(e0 & ~7, 8)
    pltpu.sync_copy(src_hbm.at[pl.ds(a_lo, SLACK)], src_buf)

    def _max_body(i, m):
        return jnp.maximum(m, as_buf[pl.ds(i * LANES, LANES)])

    m0 = lax.fori_loop(0, NP // LANES, _max_body,
                       jnp.full((LANES,), -3e38, jnp.float32))
    amax = _hmax(m0)

    iota = lax.iota(jnp.int32, LANES)
    zero = jnp.zeros((LANES,), jnp.float32)

    def _zero_body(i, _):
        out_buf[pl.ds(i * LANES, LANES)] = zero
        return 0

    lax.fori_loop(0, R * HP // LANES, _zero_body, 0)

    def _zero_den(i, _):
        den_buf[pl.ds(i * LANES, LANES)] = zero
        return 0

    lax.fori_loop(0, R, _zero_den, 0)

    def _advance(row, start):
        rs = jnp.minimum(row, R - 1)
        ov = offs_buf[pl.ds(rs, LANES)]
        nstart = start + LANES
        same = nstart < ov[1]
        nrow = jnp.where(same, rs, jnp.where(ov[2] > ov[1], rs + 1, R))
        nst = jnp.where(same, nstart, ov[1])
        nrow = jnp.where(row >= R, R, nrow)
        return nrow, nst

    def _chunk_sidx(start):
        idxv = jnp.minimum(jnp.maximum(start - a_lo, 0) + iota, SLACK - 1)
        sidx = plsc.load_gather(src_buf, [idxv])
        return jnp.clip(sidx, 0, NP - 1)

    def _fire(row, start, pb):
        sidx = _chunk_sidx(start)

        @pl.when(row < R)
        def _():
            pltpu.async_copy(h_hbm.at[sidx], h_stage.at[pb], sem.at[pb])

    c_row = jnp.int32(0)
    c_start = e0
    f_row = c_row
    f_start = c_start
    for i in range(NBUF - 1):
        _fire(f_row, f_start, jnp.int32(i))
        f_row, f_start = _advance(f_row, f_start)

    def _cond(carry):
        return carry[1] < R

    def _body(carry):
        cnt, c_row, c_start, f_row, f_start = carry
        _fire(f_row, f_start, (cnt + (NBUF - 1)) & (NBUF - 1))
        nf_row, nf_start = _advance(f_row, f_start)
        pc = cnt & (NBUF - 1)
        ov = offs_buf[pl.ds(c_row, LANES)]
        e_hi = ov[1]
        ad_r = ad_buf[pl.ds(c_row, LANES)][0]
        t = amax + ad_r
        shift = jnp.maximum(t, 0.2 * t)
        sidx = _chunk_sidx(c_start)
        av = plsc.load_gather(as_buf, [sidx])
        t2 = av + ad_r
        lg = jnp.maximum(t2, 0.2 * t2)
        ex = jnp.where(c_start + iota < e_hi, jnp.exp(lg - shift), 0.0)
        pltpu.make_async_copy(h_hbm.at[pl.ds(0, LANES)], h_stage.at[pc],
                              sem.at[pc]).wait()
        ob = c_row * HP
        accs = [out_buf[pl.ds(ob + k * LANES, LANES)] for k in range(KV)]
        for j in range(LANES):
            cv = jnp.full((LANES,), ex[j], jnp.float32)
            for k in range(KV):
                accs[k] = accs[k] + cv * h_stage[pc, j, pl.ds(k * LANES, LANES)]
        for k in range(KV):
            out_buf[pl.ds(ob + k * LANES, LANES)] = accs[k]
        db = c_row * LANES
        den_buf[pl.ds(db, LANES)] = den_buf[pl.ds(db, LANES)] + ex
        nc_row, nc_start = _advance(c_row, c_start)
        return (cnt + 1, nc_row, nc_start, nf_row, nf_start)

    lax.while_loop(_cond, _body,
                   (jnp.int32(0), c_row, c_start, f_row, f_start))

    def _norm_body(rr, _):
        d = _hsum(den_buf[pl.ds(rr * LANES, LANES)])
        denv = jnp.full((LANES,), d, jnp.float32) + 1e-16
        rinv = jnp.ones((LANES,), jnp.float32) / denv
        ob = rr * HP
        for k in range(KV):
            out_buf[pl.ds(ob + k * LANES, LANES)] = (
                out_buf[pl.ds(ob + k * LANES, LANES)] * rinv)
        return 0

    lax.fori_loop(0, R, _norm_body, 0)
    pltpu.sync_copy(out_buf, out_hbm.at[pl.ds(r0 * HP, R * HP)])


_gat_edge_built = None


def _gat_edge(*args):
    global _gat_edge_built
    if _gat_edge_built is None:
        mesh = plsc.VectorSubcoreMesh(core_axis_name="c", subcore_axis_name="s",
                                      num_cores=2, num_subcores=16)
        _gat_edge_built = functools.partial(
            pl.kernel,
            out_type=jax.ShapeDtypeStruct((NP * HP,), jnp.float32),
            mesh=mesh,
            compiler_params=pltpu.CompilerParams(needs_layout_passes=False,
                                                 use_tc_tiling_on_sc=False),
            scratch_types=[
                pltpu.VMEM((NP,), jnp.float32),
                pltpu.VMEM((OFS_T,), jnp.float32),
                pltpu.VMEM((SLACK,), jnp.int32),
                pltpu.VMEM((OFS_T,), jnp.int32),
                pltpu.VMEM((R * HP,), jnp.float32),
                pltpu.VMEM((R * LANES,), jnp.float32),
                pltpu.VMEM((NBUF, LANES, HP), jnp.float32),
                pltpu.SemaphoreType.DMA((NBUF,)),
            ],
        )(_gat_edge_body)
    return _gat_edge_built(*args)


def _pad_w(w):
    return jnp.zeros((HP, HP), jnp.float32).at[:w.shape[0], :w.shape[1]].set(w)


def _pad_a2(a_s, a_d):
    a2 = jnp.zeros((HP, 8), jnp.float32)
    return a2.at[:HID, 0].set(a_s).at[:HID, 1].set(a_d)


def _pad_b(b):
    return jnp.tile(jnp.pad(b, (0, HP - HID))[None, :], (8, 1))


def kernel(x_static, x_dynamic, x_prog, x_time, edge_index,
           W1, asrc1, adst1, bias1, W2, asrc2, adst2, bias2,
           W3, asrc3, adst3, bias3, W4, asrc4, adst4, bias4,
           W5, asrc5, adst5, bias5, fc1_W, fc1_b, fc2_W, fc2_b):
    f32 = jnp.float32
    x0 = jnp.concatenate(
        [x_static, x_dynamic, x_prog, x_time.astype(f32)], axis=-1)
    din0 = 48
    x0 = jnp.pad(x0, ((0, NP - N), (0, din0 - x0.shape[1])))
    src, dst = edge_index[0], edge_index[1]
    loop = jnp.arange(N, dtype=src.dtype)
    d_all = jnp.concatenate([dst, loop])
    s_all = jnp.concatenate([src, loop])
    d_s, s_s = lax.sort((d_all, s_all), num_keys=1)
    counts = jnp.zeros((NP,), jnp.int32).at[d_all].add(1, mode='drop')
    c = jnp.cumsum(counts)
    offs = jnp.concatenate(
        [jnp.zeros((1,), jnp.int32), c,
         jnp.full((OFFP - NP - 1,), ET, jnp.int32)]).astype(jnp.int32)
    s_pad = jnp.pad(s_s.astype(jnp.int32), (0, SLACK))

    w1p = jnp.zeros((din0, HP), f32).at[:x_static.shape[1] + x_dynamic.shape[1]
                                        + x_prog.shape[1] + x_time.shape[1],
                                        :HID].set(W1)
    layers = [
        (w1p, _pad_a2(asrc1, adst1), None),
        (_pad_w(W2), _pad_a2(asrc2, adst2), _pad_b(bias1)),
        (_pad_w(W3), _pad_a2(asrc3, adst3), _pad_b(bias2)),
        (_pad_w(W4), _pad_a2(asrc4, adst4), _pad_b(bias3)),
        (_pad_w(W5), _pad_a2(asrc5, adst5), _pad_b(bias3)),
    ]

    def run_layer(x_or_o, w, a2, b_prev, first):
        if first:
            h, aa = _tc_first(x_or_o, w, a2)
        else:
            h, aa = _tc_mid(x_or_o, b_prev, w, a2)
        adv = jnp.pad(aa[:, 1], (0, OFS_T))
        out_flat = _gat_edge(h, aa[:, 0], adv, s_pad, offs)
        return out_flat.reshape(NP, HP)

    o1 = run_layer(x0, *layers[0][:2], None, True)
    o2 = run_layer(o1, *layers[1][:2], layers[1][2], False)
    o3 = run_layer(o2, *layers[2][:2], layers[2][2], False)
    o4 = run_layer(o3, *layers[3][:2], layers[3][2], False)
    o5 = run_layer(o3, *layers[4][:2], layers[4][2], False)

    w1h = jnp.zeros((HP, 128), f32).at[:HID, :7].set(fc1_W)
    c1h = jnp.tile(jnp.pad(fc1_b, (0, 128 - 7))[None, :], (8, 1))
    w2h = jnp.zeros((HP, 128), f32).at[:HID, :3].set(fc2_W)
    c2h = jnp.tile(jnp.pad(fc2_b, (0, 128 - 3))[None, :], (8, 1))
    y1, y2 = _tc_head(o4, _pad_b(bias4), w1h, c1h,
                      o5, _pad_b(bias5), w2h, c2h)
    zero = jnp.float32(0.0)
    return (y1[:N, :7], y2[:N, :3], zero, zero)


# speedup vs baseline: 30.1910x; 1.1139x over previous
import functools

import jax
import jax.numpy as jnp
from jax import lax
from jax.experimental import pallas as pl
from jax.experimental.pallas import tpu as pltpu
from jax.experimental.pallas import tpu_sc as plsc

N = 10000
E = 320000
ET = E + N
HID = 172
HP = 176
NW = 32
R = 320
NP = NW * R
OFS_T = R + 16
OFFP = NP + 16
SLACK = 24576
ESL = ET + SLACK
LANES = 16
KV = HP // LANES


def _mm_first_body(x_ref, w_ref, a_ref, h_ref, aa_ref):
    h = jnp.dot(x_ref[...], w_ref[...], preferred_element_type=jnp.float32)
    h_ref[...] = h
    aa_ref[...] = jnp.dot(h, a_ref[...], preferred_element_type=jnp.float32)


def _mm_mid_body(o_ref, b_ref, w_ref, a_ref, h_ref, aa_ref):
    x = jnp.maximum(o_ref[...] + b_ref[0:1, :], 0.0)
    h = jnp.dot(x, w_ref[...], preferred_element_type=jnp.float32)
    h_ref[...] = h
    aa_ref[...] = jnp.dot(h, a_ref[...], preferred_element_type=jnp.float32)


def _fc_body(o4_ref, b4_ref, w1_ref, c1_ref, o5_ref, b5_ref, w2_ref, c2_ref,
             y1_ref, y2_ref):
    x4 = jnp.maximum(o4_ref[...] + b4_ref[0:1, :], 0.0)
    y1_ref[...] = jnp.dot(x4, w1_ref[...],
                          preferred_element_type=jnp.float32) + c1_ref[0:1, :]
    x5 = jnp.maximum(o5_ref[...] + b5_ref[0:1, :], 0.0)
    y2_ref[...] = jnp.dot(x5, w2_ref[...],
                          preferred_element_type=jnp.float32) + c2_ref[0:1, :]


_BM = 1024


def _tc_first(x, w, a2):
    din = x.shape[1]
    return pl.pallas_call(
        _mm_first_body,
        grid=(NP // _BM,),
        in_specs=[
            pl.BlockSpec((_BM, din), lambda i: (i, 0)),
            pl.BlockSpec((din, HP), lambda i: (0, 0)),
            pl.BlockSpec((HP, 8), lambda i: (0, 0)),
        ],
        out_specs=[
            pl.BlockSpec((_BM, HP), lambda i: (i, 0)),
            pl.BlockSpec((_BM, 8), lambda i: (i, 0)),
        ],
        out_shape=[
            jax.ShapeDtypeStruct((NP, HP), jnp.float32),
            jax.ShapeDtypeStruct((NP, 8), jnp.float32),
        ],
    )(x, w, a2)


def _tc_mid(o_prev, b_prev, w, a2):
    return pl.pallas_call(
        _mm_mid_body,
        grid=(NP // _BM,),
        in_specs=[
            pl.BlockSpec((_BM, HP), lambda i: (i, 0)),
            pl.BlockSpec((8, HP), lambda i: (0, 0)),
            pl.BlockSpec((HP, HP), lambda i: (0, 0)),
            pl.BlockSpec((HP, 8), lambda i: (0, 0)),
        ],
        out_specs=[
            pl.BlockSpec((_BM, HP), lambda i: (i, 0)),
            pl.BlockSpec((_BM, 8), lambda i: (i, 0)),
        ],
        out_shape=[
            jax.ShapeDtypeStruct((NP, HP), jnp.float32),
            jax.ShapeDtypeStruct((NP, 8), jnp.float32),
        ],
    )(o_prev, b_prev, w, a2)


def _tc_head(o4, b4, w1, c1, o5, b5, w2, c2):
    return pl.pallas_call(
        _fc_body,
        grid=(NP // _BM,),
        in_specs=[
            pl.BlockSpec((_BM, HP), lambda i: (i, 0)),
            pl.BlockSpec((8, HP), lambda i: (0, 0)),
            pl.BlockSpec((HP, 128), lambda i: (0, 0)),
            pl.BlockSpec((8, 128), lambda i: (0, 0)),
            pl.BlockSpec((_BM, HP), lambda i: (i, 0)),
            pl.BlockSpec((8, HP), lambda i: (0, 0)),
            pl.BlockSpec((HP, 128), lambda i: (0, 0)),
            pl.BlockSpec((8, 128), lambda i: (0, 0)),
        ],
        out_specs=[
            pl.BlockSpec((_BM, 128), lambda i: (i, 0)),
            pl.BlockSpec((_BM, 128), lambda i: (i, 0)),
        ],
        out_shape=[
            jax.ShapeDtypeStruct((NP, 128), jnp.float32),
            jax.ShapeDtypeStruct((NP, 128), jnp.float32),
        ],
    )(o4, b4, w1, c1, o5, b5, w2, c2)


def _hmax(v):
    r = v[0]
    for j in range(1, LANES):
        r = jnp.maximum(r, v[j])
    return r


def _hsum(v):
    r = v[0]
    for j in range(1, LANES):
        r = r + v[j]
    return r


NBUF = 8


def _gat_edge_body(h_hbm, as_hbm, ad_hbm, src_hbm, offs_hbm, out_hbm,
                   as_buf, ad_buf, src_buf, offs_buf, out_buf, den_buf,
                   h_stage, sem):
    wid = lax.axis_index("s") * 2 + lax.axis_index("c")
    r0 = wid * R
    pltpu.sync_copy(as_hbm, as_buf)
    pltpu.sync_copy(ad_hbm.at[pl.ds(r0, OFS_T)], ad_buf)
    pltpu.sync_copy(offs_hbm.at[pl.ds(r0, OFS_T)], offs_buf)
    e0 = offs_buf[pl.ds(0, LANES)][0]
    a_lo = pl.multiple_of(e0 & ~7, 8)
    pltpu.sync_copy(src_hbm.at[pl.ds(a_lo, SLACK)], src_buf)

    def _max_body(i, m):
        return jnp.maximum(m, as_buf[pl.ds(i * LANES, LANES)])

    m0 = lax.fori_loop(0, NP // LANES, _max_body,
                       jnp.full((LANES,), -3e38, jnp.float32))
    amax = _hmax(m0)

    iota = lax.iota(jnp.int32, LANES)
    zero = jnp.zeros((LANES,), jnp.float32)

    def _zero_body(i, _):
        out_buf[pl.ds(i * LANES, LANES)] = zero
        return 0

    lax.fori_loop(0, R * HP // LANES, _zero_body, 0)

    def _zero_den(i, _):
        den_buf[pl.ds(i * LANES, LANES)] = zero
        return 0

    lax.fori_loop(0, R, _zero_den, 0)

    def _advance(row, start):
        rs = jnp.minimum(row, R - 1)
        ov = offs_buf[pl.ds(rs, LANES)]
        nstart = start + LANES
        same = nstart < ov[1]
        nrow = jnp.where(same, rs, jnp.where(ov[2] > ov[1], rs + 1, R))
        nst = jnp.where(same, nstart, ov[1])
        nrow = jnp.where(row >= R, R, nrow)
        return nrow, nst

    def _chunk_sidx(start):
        idxv = jnp.minimum(jnp.maximum(start - a_lo, 0) + iota, SLACK - 1)
        sidx = plsc.load_gather(src_buf, [idxv])
        return jnp.clip(sidx, 0, NP - 1)

    def _fire(row, start, pb):
        sidx = _chunk_sidx(start)

        @pl.when(row < R)
        def _():
            pltpu.async_copy(h_hbm.at[sidx], h_stage.at[pb], sem.at[pb])

    c_row = jnp.int32(0)
    c_start = e0
    f_row = c_row
    f_start = c_start
    for i in range(NBUF - 1):
        _fire(f_row, f_start, jnp.int32(i))
        f_row, f_start = _advance(f_row, f_start)

    def _cond(carry):
        return carry[1] < R

    def _body(carry):
        cnt, c_row, c_start, f_row, f_start = carry
        _fire(f_row, f_start, (cnt + (NBUF - 1)) & (NBUF - 1))
        nf_row, nf_start = _advance(f_row, f_start)
        pc = cnt & (NBUF - 1)
        ov = offs_buf[pl.ds(c_row, LANES)]
        e_hi = ov[1]
        ad_r = ad_buf[pl.ds(c_row, LANES)][0]
        t = amax + ad_r
        shift = jnp.maximum(t, 0.2 * t)
        sidx = _chunk_sidx(c_start)
        av = plsc.load_gather(as_buf, [sidx])
        t2 = av + ad_r
        lg = jnp.maximum(t2, 0.2 * t2)
        ex = jnp.where(c_start + iota < e_hi, jnp.exp(lg - shift), 0.0)
        pltpu.make_async_copy(h_hbm.at[pl.ds(0, LANES)], h_stage.at[pc],
                              sem.at[pc]).wait()
        ob = c_row * HP
        accs = [out_buf[pl.ds(ob + k * LANES, LANES)] for k in range(KV)]
        for j in range(LANES):
            cv = jnp.full((LANES,), ex[j], jnp.float32)
            for k in range(KV):
                accs[k] = accs[k] + cv * h_stage[pc, j, pl.ds(k * LANES, LANES)]
        for k in range(KV):
            out_buf[pl.ds(ob + k * LANES, LANES)] = accs[k]
        db = c_row * LANES
        den_buf[pl.ds(db, LANES)] = den_buf[pl.ds(db, LANES)] + ex
        nc_row, nc_start = _advance(c_row, c_start)
        return (cnt + 1, nc_row, nc_start, nf_row, nf_start)

    lax.while_loop(_cond, _body,
                   (jnp.int32(0), c_row, c_start, f_row, f_start))

    def _norm_body(rr, _):
        d = _hsum(den_buf[pl.ds(rr * LANES, LANES)])
        denv = jnp.full((LANES,), d, jnp.float32) + 1e-16
        rinv = jnp.ones((LANES,), jnp.float32) / denv
        ob = rr * HP
        for k in range(KV):
            out_buf[pl.ds(ob + k * LANES, LANES)] = (
                out_buf[pl.ds(ob + k * LANES, LANES)] * rinv)
        return 0

    lax.fori_loop(0, R, _norm_body, 0)
    pltpu.sync_copy(out_buf, out_hbm.at[pl.ds(r0 * HP, R * HP)])


_gat_edge_built = None


def _gat_edge(*args):
    global _gat_edge_built
    if _gat_edge_built is None:
        mesh = plsc.VectorSubcoreMesh(core_axis_name="c", subcore_axis_name="s",
                                      num_cores=2, num_subcores=16)
        _gat_edge_built = functools.partial(
            pl.kernel,
            out_type=jax.ShapeDtypeStruct((NP * HP,), jnp.float32),
            mesh=mesh,
            compiler_params=pltpu.CompilerParams(needs_layout_passes=False,
                                                 use_tc_tiling_on_sc=False),
            scratch_types=[
                pltpu.VMEM((NP,), jnp.float32),
                pltpu.VMEM((OFS_T,), jnp.float32),
                pltpu.VMEM((SLACK,), jnp.int32),
                pltpu.VMEM((OFS_T,), jnp.int32),
                pltpu.VMEM((R * HP,), jnp.float32),
                pltpu.VMEM((R * LANES,), jnp.float32),
                pltpu.VMEM((NBUF, LANES, HP), jnp.float32),
                pltpu.SemaphoreType.DMA((NBUF,)),
            ],
        )(_gat_edge_body)
    return _gat_edge_built(*args)


def _pad_w(w):
    return jnp.zeros((HP, HP), jnp.float32).at[:w.shape[0], :w.shape[1]].set(w)


def _pad_a2(a_s, a_d):
    a2 = jnp.zeros((HP, 8), jnp.float32)
    return a2.at[:HID, 0].set(a_s).at[:HID, 1].set(a_d)


def _pad_b(b):
    return jnp.tile(jnp.pad(b, (0, HP - HID))[None, :], (8, 1))


def kernel(x_static, x_dynamic, x_prog, x_time, edge_index,
           W1, asrc1, adst1, bias1, W2, asrc2, adst2, bias2,
           W3, asrc3, adst3, bias3, W4, asrc4, adst4, bias4,
           W5, asrc5, adst5, bias5, fc1_W, fc1_b, fc2_W, fc2_b):
    f32 = jnp.float32
    x0 = jnp.concatenate(
        [x_static, x_dynamic, x_prog, x_time.astype(f32)], axis=-1)
    din0 = 48
    x0 = jnp.pad(x0, ((0, NP - N), (0, din0 - x0.shape[1])))
    src, dst = edge_index[0], edge_index[1]
    loop = jnp.arange(N, dtype=src.dtype)
    d_all = jnp.concatenate([dst, loop])
    s_all = jnp.concatenate([src, loop])
    d_s, s_s = lax.sort((d_all, s_all), num_keys=1)
    counts = jnp.zeros((NP,), jnp.int32).at[d_all].add(1, mode='drop')
    c = jnp.cumsum(counts)
    offs = jnp.concatenate(
        [jnp.zeros((1,), jnp.int32), c,
         jnp.full((OFFP - NP - 1,), ET, jnp.int32)]).astype(jnp.int32)
    s_pad = jnp.pad(s_s.astype(jnp.int32), (0, SLACK))

    w1p = jnp.zeros((din0, HP), f32).at[:x_static.shape[1] + x_dynamic.shape[1]
                                        + x_prog.shape[1] + x_time.shape[1],
                                        :HID].set(W1)
    layers = [
        (w1p, _pad_a2(asrc1, adst1), None),
        (_pad_w(W2), _pad_a2(asrc2, adst2), _pad_b(bias1)),
        (_pad_w(W3), _pad_a2(asrc3, adst3), _pad_b(bias2)),
        (_pad_w(W4), _pad_a2(asrc4, adst4), _pad_b(bias3)),
        (_pad_w(W5), _pad_a2(asrc5, adst5), _pad_b(bias3)),
    ]

    def run_layer(x_or_o, w, a2, b_prev, first):
        if first:
            h, aa = _tc_first(x_or_o, w, a2)
        else:
            h, aa = _tc_mid(x_or_o, b_prev, w, a2)
        adv = jnp.pad(aa[:, 1], (0, OFS_T))
        out_flat = _gat_edge(h, aa[:, 0], adv, s_pad, offs)
        return out_flat.reshape(NP, HP)

    o1 = run_layer(x0, *layers[0][:2], None, True)
    o2 = run_layer(o1, *layers[1][:2], layers[1][2], False)
    o3 = run_layer(o2, *layers[2][:2], layers[2][2], False)
    o4 = run_layer(o3, *layers[3][:2], layers[3][2], False)
    o5 = run_layer(o3, *layers[4][:2], layers[4][2], False)

    w1h = jnp.zeros((HP, 128), f32).at[:HID, :7].set(fc1_W)
    c1h = jnp.tile(jnp.pad(fc1_b, (0, 128 - 7))[None, :], (8, 1))
    w2h = jnp.zeros((HP, 128), f32).at[:HID, :3].set(fc2_W)
    c2h = jnp.tile(jnp.pad(fc2_b, (0, 128 - 3))[None, :], (8, 1))
    y1, y2 = _tc_head(o4, _pad_b(bias4), w1h, c1h,
                      o5, _pad_b(bias5), w2h, c2h)
    zero = jnp.float32(0.0)
    return (y1[:N, :7], y2[:N, :3], zero, zero)


# speedup vs baseline: 30.4832x; 1.0097x over previous
import functools

import numpy as np

import jax
import jax.numpy as jnp
from jax import lax
from jax.experimental import pallas as pl
from jax.experimental.pallas import tpu as pltpu
from jax.experimental.pallas import tpu_sc as plsc

N = 10000
E = 320000
ET = E + N
HID = 172
HP = 192
NW = 32
R = 320
NP = NW * R
OFS_T = R + 16
OFFP = NP + 16
SLACK = 16384
ESL = ET + SLACK
LANES = 16
KV = HP // LANES
KB = HP // 32
HW = HP // 2

IDXE = [32 * (w // 16) + (w % 16) for w in range(HW)]
IDXO = [32 * (w // 16) + 16 + (w % 16) for w in range(HW)]


def _pack_h(he, ho):
    be = lax.bitcast_convert_type(he.astype(jnp.bfloat16), jnp.uint16)
    bo = lax.bitcast_convert_type(ho.astype(jnp.bfloat16), jnp.uint16)
    return be.astype(jnp.uint32) | (bo.astype(jnp.uint32) << 16)


def _mm_first_body(x_ref, we_ref, wo_ref, wa_ref, h_ref, aa_ref):
    x = x_ref[...]
    he = jnp.dot(x, we_ref[...], preferred_element_type=jnp.float32)
    ho = jnp.dot(x, wo_ref[...], preferred_element_type=jnp.float32)
    aa_ref[...] = jnp.dot(x, wa_ref[...], preferred_element_type=jnp.float32)
    h_ref[...] = _pack_h(he, ho)


def _mm_mid_body(o_ref, b_ref, we_ref, wo_ref, wa_ref, h_ref, aa_ref):
    x = jnp.maximum(o_ref[...] + b_ref[0:1, :], 0.0)
    he = jnp.dot(x, we_ref[...], preferred_element_type=jnp.float32)
    ho = jnp.dot(x, wo_ref[...], preferred_element_type=jnp.float32)
    aa_ref[...] = jnp.dot(x, wa_ref[...], preferred_element_type=jnp.float32)
    h_ref[...] = _pack_h(he, ho)


def _fc_body(o4_ref, b4_ref, w1_ref, c1_ref, o5_ref, b5_ref, w2_ref, c2_ref,
             y1_ref, y2_ref):
    x4 = jnp.maximum(o4_ref[...] + b4_ref[0:1, :], 0.0)
    y1_ref[...] = jnp.dot(x4, w1_ref[...],
                          preferred_element_type=jnp.float32) + c1_ref[0:1, :]
    x5 = jnp.maximum(o5_ref[...] + b5_ref[0:1, :], 0.0)
    y2_ref[...] = jnp.dot(x5, w2_ref[...],
                          preferred_element_type=jnp.float32) + c2_ref[0:1, :]


_BM = 1024


def _tc_first(x, we, wo, wa):
    din = x.shape[1]
    return pl.pallas_call(
        _mm_first_body,
        grid=(NP // _BM,),
        in_specs=[
            pl.BlockSpec((_BM, din), lambda i: (i, 0)),
            pl.BlockSpec((din, HW), lambda i: (0, 0)),
            pl.BlockSpec((din, HW), lambda i: (0, 0)),
            pl.BlockSpec((din, 8), lambda i: (0, 0)),
        ],
        out_specs=[
            pl.BlockSpec((_BM, HW), lambda i: (i, 0)),
            pl.BlockSpec((_BM, 8), lambda i: (i, 0)),
        ],
        out_shape=[
            jax.ShapeDtypeStruct((NP, HW), jnp.uint32),
            jax.ShapeDtypeStruct((NP, 8), jnp.float32),
        ],
    )(x, we, wo, wa)


def _tc_mid(o_prev, b_prev, we, wo, wa):
    return pl.pallas_call(
        _mm_mid_body,
        grid=(NP // _BM,),
        in_specs=[
            pl.BlockSpec((_BM, HP), lambda i: (i, 0)),
            pl.BlockSpec((8, HP), lambda i: (0, 0)),
            pl.BlockSpec((HP, HW), lambda i: (0, 0)),
            pl.BlockSpec((HP, HW), lambda i: (0, 0)),
            pl.BlockSpec((HP, 8), lambda i: (0, 0)),
        ],
        out_specs=[
            pl.BlockSpec((_BM, HW), lambda i: (i, 0)),
            pl.BlockSpec((_BM, 8), lambda i: (i, 0)),
        ],
        out_shape=[
            jax.ShapeDtypeStruct((NP, HW), jnp.uint32),
            jax.ShapeDtypeStruct((NP, 8), jnp.float32),
        ],
    )(o_prev, b_prev, we, wo, wa)


def _tc_head(o4, b4, w1, c1, o5, b5, w2, c2):
    return pl.pallas_call(
        _fc_body,
        grid=(NP // _BM,),
        in_specs=[
            pl.BlockSpec((_BM, HP), lambda i: (i, 0)),
            pl.BlockSpec((8, HP), lambda i: (0, 0)),
            pl.BlockSpec((HP, 128), lambda i: (0, 0)),
            pl.BlockSpec((8, 128), lambda i: (0, 0)),
            pl.BlockSpec((_BM, HP), lambda i: (i, 0)),
            pl.BlockSpec((8, HP), lambda i: (0, 0)),
            pl.BlockSpec((HP, 128), lambda i: (0, 0)),
            pl.BlockSpec((8, 128), lambda i: (0, 0)),
        ],
        out_specs=[
            pl.BlockSpec((_BM, 128), lambda i: (i, 0)),
            pl.BlockSpec((_BM, 128), lambda i: (i, 0)),
        ],
        out_shape=[
            jax.ShapeDtypeStruct((NP, 128), jnp.float32),
            jax.ShapeDtypeStruct((NP, 128), jnp.float32),
        ],
    )(o4, b4, w1, c1, o5, b5, w2, c2)


def _hmax(v):
    r = v[0]
    for j in range(1, LANES):
        r = jnp.maximum(r, v[j])
    return r


def _hsum(v):
    r = v[0]
    for j in range(1, LANES):
        r = r + v[j]
    return r


NBUF = 16


def _gat_edge_body(h_hbm, as_hbm, ad_hbm, src_hbm, offs_hbm, out_hbm,
                   as_buf, ad_buf, src_buf, offs_buf, out_buf, den_buf,
                   h_stage, sem):
    wid = lax.axis_index("s") * 2 + lax.axis_index("c")
    r0 = wid * R
    pltpu.sync_copy(as_hbm, as_buf)
    pltpu.sync_copy(ad_hbm.at[pl.ds(r0, OFS_T)], ad_buf)
    pltpu.sync_copy(offs_hbm.at[pl.ds(r0, OFS_T)], offs_buf)
    e0 = offs_buf[pl.ds(0, LANES)][0]
    a_lo = pl.multiple_of(e0 & ~7, 8)
    pltpu.sync_copy(src_hbm.at[pl.ds(a_lo, SLACK)], src_buf)

    def _max_body(i, m):
        return jnp.maximum(m, as_buf[pl.ds(i * LANES, LANES)])

    m0 = lax.fori_loop(0, NP // LANES, _max_body,
                       jnp.full((LANES,), -3e38, jnp.float32))
    amax = _hmax(m0)

    iota = lax.iota(jnp.int32, LANES)
    zero = jnp.zeros((LANES,), jnp.float32)

    def _zero_body(i, _):
        out_buf[pl.ds(i * LANES, LANES)] = zero
        return 0

    lax.fori_loop(0, R * HP // LANES, _zero_body, 0)

    def _zero_den(i, _):
        den_buf[pl.ds(i * LANES, LANES)] = zero
        return 0

    lax.fori_loop(0, R, _zero_den, 0)

    def _advance(row, start):
        rs = jnp.minimum(row, R - 1)
        ov = offs_buf[pl.ds(rs, LANES)]
        nstart = start + LANES
        same = nstart < ov[1]
        nrow = jnp.where(same, rs, jnp.where(ov[2] > ov[1], rs + 1, R))
        nst = jnp.where(same, nstart, ov[1])
        nrow = jnp.where(row >= R, R, nrow)
        return nrow, nst

    def _chunk_sidx(start):
        idxv = jnp.minimum(jnp.maximum(start - a_lo, 0) + iota, SLACK - 1)
        sidx = plsc.load_gather(src_buf, [idxv])
        return jnp.clip(sidx, 0, NP - 1)

    def _fire(row, start, pb):
        sidx = _chunk_sidx(start)

        @pl.when(row < R)
        def _():
            pltpu.async_copy(h_hbm.at[sidx], h_stage.at[pb], sem.at[pb])

    c_row = jnp.int32(0)
    c_start = e0
    f_row = c_row
    f_start = c_start
    for i in range(NBUF - 1):
        _fire(f_row, f_start, jnp.int32(i))
        f_row, f_start = _advance(f_row, f_start)

    def _cond(carry):
        return carry[1] < R

    def _body(carry):
        cnt, c_row, c_start, f_row, f_start = carry
        _fire(f_row, f_start, (cnt + (NBUF - 1)) & (NBUF - 1))
        nf_row, nf_start = _advance(f_row, f_start)
        pc = cnt & (NBUF - 1)
        ov = offs_buf[pl.ds(c_row, LANES)]
        e_hi = ov[1]
        ad_r = ad_buf[pl.ds(c_row, LANES)][0]
        t = amax + ad_r
        shift = jnp.maximum(t, 0.2 * t)
        sidx = _chunk_sidx(c_start)
        av = plsc.load_gather(as_buf, [sidx])
        t2 = av + ad_r
        lg = jnp.maximum(t2, 0.2 * t2)
        ex = jnp.where(c_start + iota < e_hi, jnp.exp(lg - shift), 0.0)
        pltpu.make_async_copy(h_hbm.at[pl.ds(0, LANES)], h_stage.at[pc],
                              sem.at[pc]).wait()
        ob = c_row * HP
        accs = [out_buf[pl.ds(ob + k * LANES, LANES)] for k in range(KV)]
        for j in range(LANES):
            cv = jnp.full((LANES,), ex[j], jnp.float32)
            for m in range(KB):
                v = plsc.bitcast(h_stage[pc, j, pl.ds(m * LANES, LANES)],
                                 jnp.bfloat16)
                va, vb = plsc.unpack(v, format=plsc.PackFormat.INTERLEAVED)
                accs[2 * m] = accs[2 * m] + cv * va
                accs[2 * m + 1] = accs[2 * m + 1] + cv * vb
        for k in range(KV):
            out_buf[pl.ds(ob + k * LANES, LANES)] = accs[k]
        db = c_row * LANES
        den_buf[pl.ds(db, LANES)] = den_buf[pl.ds(db, LANES)] + ex
        nc_row, nc_start = _advance(c_row, c_start)
        return (cnt + 1, nc_row, nc_start, nf_row, nf_start)

    lax.while_loop(_cond, _body,
                   (jnp.int32(0), c_row, c_start, f_row, f_start))

    def _norm_body(rr, _):
        d = _hsum(den_buf[pl.ds(rr * LANES, LANES)])
        denv = jnp.full((LANES,), d, jnp.float32) + 1e-16
        rinv = jnp.ones((LANES,), jnp.float32) / denv
        ob = rr * HP
        for k in range(KV):
            out_buf[pl.ds(ob + k * LANES, LANES)] = (
                out_buf[pl.ds(ob + k * LANES, LANES)] * rinv)
        return 0

    lax.fori_loop(0, R, _norm_body, 0)
    pltpu.sync_copy(out_buf, out_hbm.at[pl.ds(r0 * HP, R * HP)])


_gat_edge_built = None


def _gat_edge(*args):
    global _gat_edge_built
    if _gat_edge_built is None:
        mesh = plsc.VectorSubcoreMesh(core_axis_name="c", subcore_axis_name="s",
                                      num_cores=2, num_subcores=16)
        _gat_edge_built = functools.partial(
            pl.kernel,
            out_type=jax.ShapeDtypeStruct((NP * HP,), jnp.float32),
            mesh=mesh,
            compiler_params=pltpu.CompilerParams(needs_layout_passes=False,
                                                 use_tc_tiling_on_sc=False),
            scratch_types=[
                pltpu.VMEM((NP,), jnp.float32),
                pltpu.VMEM((OFS_T,), jnp.float32),
                pltpu.VMEM((SLACK,), jnp.int32),
                pltpu.VMEM((OFS_T,), jnp.int32),
                pltpu.VMEM((R * HP,), jnp.float32),
                pltpu.VMEM((R * LANES,), jnp.float32),
                pltpu.VMEM((NBUF, LANES, HW), jnp.uint32),
                pltpu.SemaphoreType.DMA((NBUF,)),
            ],
        )(_gat_edge_body)
    return _gat_edge_built(*args)


_IDXE_A = np.array(IDXE)
_IDXO_A = np.array(IDXO)


def _prep_w(w, a_s, a_d):
    din = HP if w.shape[0] > 48 else 48
    wp = jnp.zeros((din, HP), jnp.float32).at[:w.shape[0], :w.shape[1]].set(w)
    a2 = jnp.zeros((HP, 8), jnp.float32)
    a2 = a2.at[:HID, 0].set(a_s).at[:HID, 1].set(a_d)
    return wp[:, _IDXE_A], wp[:, _IDXO_A], wp @ a2


def _pad_b(b):
    return jnp.tile(jnp.pad(b, (0, HP - HID))[None, :], (8, 1))


def kernel(x_static, x_dynamic, x_prog, x_time, edge_index,
           W1, asrc1, adst1, bias1, W2, asrc2, adst2, bias2,
           W3, asrc3, adst3, bias3, W4, asrc4, adst4, bias4,
           W5, asrc5, adst5, bias5, fc1_W, fc1_b, fc2_W, fc2_b):
    f32 = jnp.float32
    x0 = jnp.concatenate(
        [x_static, x_dynamic, x_prog, x_time.astype(f32)], axis=-1)
    din0 = 48
    x0 = jnp.pad(x0, ((0, NP - N), (0, din0 - x0.shape[1])))
    src, dst = edge_index[0], edge_index[1]
    loop = jnp.arange(N, dtype=src.dtype)
    d_all = jnp.concatenate([dst, loop])
    s_all = jnp.concatenate([src, loop])
    d_s, s_s = lax.sort((d_all, s_all), num_keys=1)
    counts = jnp.zeros((NP,), jnp.int32).at[d_all].add(1, mode='drop')
    c = jnp.cumsum(counts)
    offs = jnp.concatenate(
        [jnp.zeros((1,), jnp.int32), c,
         jnp.full((OFFP - NP - 1,), ET, jnp.int32)]).astype(jnp.int32)
    s_pad = jnp.pad(s_s.astype(jnp.int32), (0, SLACK))

    layers = [
        (_prep_w(W1, asrc1, adst1), None),
        (_prep_w(W2, asrc2, adst2), _pad_b(bias1)),
        (_prep_w(W3, asrc3, adst3), _pad_b(bias2)),
        (_prep_w(W4, asrc4, adst4), _pad_b(bias3)),
        (_prep_w(W5, asrc5, adst5), _pad_b(bias3)),
    ]

    def run_layer(x_or_o, wts, b_prev, first):
        we, wo, wa = wts
        if first:
            h, aa = _tc_first(x_or_o, we, wo, wa)
        else:
            h, aa = _tc_mid(x_or_o, b_prev, we, wo, wa)
        adv = jnp.pad(aa[:, 1], (0, OFS_T))
        out_flat = _gat_edge(h, aa[:, 0], adv, s_pad, offs)
        return out_flat.reshape(NP, HP)

    o1 = run_layer(x0, layers[0][0], None, True)
    o2 = run_layer(o1, layers[1][0], layers[1][1], False)
    o3 = run_layer(o2, layers[2][0], layers[2][1], False)
    o4 = run_layer(o3, layers[3][0], layers[3][1], False)
    o5 = run_layer(o3, layers[4][0], layers[4][1], False)

    w1h = jnp.zeros((HP, 128), f32).at[:HID, :7].set(fc1_W)
    c1h = jnp.tile(jnp.pad(fc1_b, (0, 128 - 7))[None, :], (8, 1))
    w2h = jnp.zeros((HP, 128), f32).at[:HID, :3].set(fc2_W)
    c2h = jnp.tile(jnp.pad(fc2_b, (0, 128 - 3))[None, :], (8, 1))
    y1, y2 = _tc_head(o4, _pad_b(bias4), w1h, c1h,
                      o5, _pad_b(bias5), w2h, c2h)
    zero = jnp.float32(0.0)
    return (y1[:N, :7], y2[:N, :3], zero, zero)
